# baseline (speedup 1.0000x reference)
import jax
import jax.numpy as jnp
from jax.experimental import pallas as pl
from jax.experimental.pallas import tpu as pltpu
from jax.experimental.pallas import tpu_sc as plsc

_N = 10000
_NPAD = 10240
_E = 320000
_DIN = 128
_H1 = 640
_H2 = 320
_DOUT = 128

_NC = 2
_NS = 16
_C = 20
_EPT = _E // (_NC * _NS)
_NCH = _EPT // _C
_NBC = 100
_NRING = 4
_NSB = _NCH // _NBC
_RPT = _NPAD // _NS
_DC = 128

_F32 = jnp.float32


def _make_sc_segsum(ncc):
    mesh = plsc.VectorSubcoreMesh(core_axis_name="c", subcore_axis_name="s",
                                  num_cores=_NC, num_subcores=_NS)

    def body(t_hbm, src_hbm, dst_hbm, out_hbm,
             src_v, dst_v, rows0, rows1, rows2, rows3, acc,
             sem0, sem1, sem2, sem3):
        cid = jax.lax.axis_index("c")
        sid = jax.lax.axis_index("s")

        rows = [rows0, rows1, rows2, rows3]
        sems = [sem0, sem1, sem2, sem3]

        def zb(k, carry):
            r = k // (_DC // 16)
            col = (k % (_DC // 16)) * 16
            rows0[r, pl.ds(col, 16)] = jnp.zeros((16,), _F32)
            return carry

        for cc in range(ncc):
            jax.lax.fori_loop(0, _C * (_DC // 16), zb, 0)
            for k in range(_RPT // _C):
                pltpu.sync_copy(rows0, acc.at[pl.ds(sid * _RPT + k * _C, _C)])
            plsc.subcore_barrier()

            tc = t_hbm.at[cc]
            for sb in range(_NSB):
                pltpu.sync_copy(src_hbm.at[cid, sid, sb], src_v)
                pltpu.sync_copy(dst_hbm.at[cid, sid, sb], dst_v)
                for b in range(_NRING):
                    pltpu.async_copy(tc.at[src_v.at[b]], rows[b], sems[b])

                def group(g, carry):
                    for b in range(_NRING):
                        j = _NRING * g + b
                        pltpu.make_async_copy(tc.at[src_v.at[j]], rows[b],
                                              sems[b]).wait()
                        pltpu.sync_copy(rows[b], acc.at[dst_v.at[j]],
                                        add=True)

                        @pl.when(j + _NRING < _NBC)
                        def _():
                            pltpu.async_copy(tc.at[src_v.at[j + _NRING]],
                                             rows[b], sems[b])
                    return carry

                jax.lax.fori_loop(0, _NBC // _NRING, group, 0)

            plsc.subcore_barrier()
            pltpu.sync_copy(acc.at[pl.ds(sid * _RPT, _RPT)],
                            out_hbm.at[cc, cid, pl.ds(sid * _RPT, _RPT)])

    return pl.kernel(
        body,
        out_type=jax.ShapeDtypeStruct((ncc, _NC, _NPAD, _DC), _F32),
        mesh=mesh,
        scratch_types=[
            pltpu.VMEM((_NBC, _C), jnp.int32),
            pltpu.VMEM((_NBC, _C), jnp.int32),
            pltpu.VMEM((_C, _DC), _F32),
            pltpu.VMEM((_C, _DC), _F32),
            pltpu.VMEM((_C, _DC), _F32),
            pltpu.VMEM((_C, _DC), _F32),
            pltpu.VMEM_SHARED((_NPAD, _DC), _F32),
            pltpu.SemaphoreType.DMA,
            pltpu.SemaphoreType.DMA,
            pltpu.SemaphoreType.DMA,
            pltpu.SemaphoreType.DMA,
        ],
    )


_BM = 512
_GRID = (_NPAD // _BM,)


def _elu(x):
    return jnp.where(x > 0, x, jnp.exp(jnp.minimum(x, 0.0)) - 1.0)


def _dot(a, b):
    return jnp.dot(a, b, preferred_element_type=_F32)


def _full(shape):
    nd = len(shape)
    return pl.BlockSpec(shape, lambda i: (0,) * nd)


def _rows(d):
    return pl.BlockSpec((_BM, d), lambda i: (i, 0))


def _agg(ncc):
    return pl.BlockSpec((ncc, _NC, _BM, _DC), lambda i: (0, 0, i, 0))


def _k1_body(f_ref, agg_ref, w1a, b1a, w1b, b1b, w2a, wres, bres,
             u2_ref, res_ref):
    x0 = f_ref[...]
    res_ref[...] = _elu(_dot(x0, wres[...]) + bres[...])
    x = x0 + agg_ref[0, 0] + agg_ref[0, 1]
    z1 = _elu(_dot(x, w1a[...]) + b1a[...])
    x1 = _elu(_elu(_dot(z1, w1b[...]) + b1b[...]))
    u2 = _dot(x1, w2a[...])
    for c in range(3):
        u2_ref[c] = u2[:, c * _DC:(c + 1) * _DC]


def _make_k1(interpret=False):
    return pl.pallas_call(
        _k1_body,
        grid=_GRID,
        in_specs=[_rows(_DIN), _agg(1), _full((_DIN, _H1)), _full((1, _H1)),
                  _full((_H1, _H1)), _full((1, _H1)), _full((_H1, 3 * _DC)),
                  _full((_DIN, _DOUT)), _full((1, _DOUT))],
        out_specs=[pl.BlockSpec((3, _BM, _DC), lambda i: (0, i, 0)),
                   _rows(_DOUT)],
        out_shape=[jax.ShapeDtypeStruct((3, _NPAD, _DC), _F32),
                   jax.ShapeDtypeStruct((_NPAD, _DOUT), _F32)],
        interpret=interpret,
    )


def _k2_body(u2_ref, agg_ref, b2a, w2b, b2b, w3a, u3_ref):
    parts = [u2_ref[c] + agg_ref[c, 0] + agg_ref[c, 1] for c in range(3)]
    z2 = _elu(jnp.concatenate(parts, axis=1)[:, :_H2] + b2a[...])
    x2 = _elu(_elu(_dot(z2, w2b[...]) + b2b[...]))
    u3_ref[...] = _dot(x2, w3a[...])


def _make_k2(interpret=False):
    return pl.pallas_call(
        _k2_body,
        grid=_GRID,
        in_specs=[pl.BlockSpec((3, _BM, _DC), lambda i: (0, i, 0)), _agg(3),
                  _full((1, _H2)), _full((_H2, _H2)), _full((1, _H2)),
                  _full((_H2, _DOUT))],
        out_specs=[_rows(_DOUT)],
        out_shape=[jax.ShapeDtypeStruct((_NPAD, _DOUT), _F32)],
        interpret=interpret,
    )


def _k3_body(u3_ref, agg_ref, b3a, w3b, b3b, out_ref):
    z3 = _elu(u3_ref[...] + agg_ref[0, 0] + agg_ref[0, 1] + b3a[...])
    out_ref[...] = _elu(_dot(z3, w3b[...]) + b3b[...])


def _make_k3(interpret=False):
    return pl.pallas_call(
        _k3_body,
        grid=_GRID,
        in_specs=[_rows(_DOUT), _agg(1), _full((1, _DOUT)),
                  _full((_DOUT, _DOUT)), _full((1, _DOUT))],
        out_specs=[_rows(_DOUT)],
        out_shape=[jax.ShapeDtypeStruct((_NPAD, _DOUT), _F32)],
        interpret=interpret,
    )


def kernel(features, edge_index,
           W1a, b1a, W1b, b1b,
           W2a, b2a, W2b, b2b,
           W3a, b3a, W3b, b3b,
           Wres, bres):
    f = jnp.zeros((_NPAD, _DIN), _F32).at[:_N].set(features)
    src = edge_index[0].astype(jnp.int32).reshape(_NC, _NS, _NSB, _NBC, _C)
    dst = edge_index[1].astype(jnp.int32).reshape(_NC, _NS, _NSB, _NBC, _C)
    w2a_pad = jnp.zeros((_H1, 3 * _DC), _F32).at[:, :_H2].set(W2a)

    agg1 = _make_sc_segsum(1)(f.reshape(1, _NPAD, _DIN), src, dst)
    u2, res = _make_k1()(f, agg1, W1a, b1a.reshape(1, -1),
                         W1b, b1b.reshape(1, -1), w2a_pad,
                         Wres, bres.reshape(1, -1))
    agg2 = _make_sc_segsum(3)(u2, src, dst)
    (u3,) = _make_k2()(u2, agg2, b2a.reshape(1, -1),
                       W2b, b2b.reshape(1, -1), W3a)
    agg3 = _make_sc_segsum(1)(u3.reshape(1, _NPAD, _DOUT), src, dst)
    (out,) = _make_k3()(u3, agg3, b3a.reshape(1, -1),
                        W3b, b3b.reshape(1, -1))
    return (out[:_N], res[:_N])


# speedup vs baseline: 1.0402x; 1.0402x over previous
import jax
import jax.numpy as jnp
from jax.experimental import pallas as pl
from jax.experimental.pallas import tpu as pltpu
from jax.experimental.pallas import tpu_sc as plsc

_N = 10000
_NPAD = 10240
_E = 320000
_DIN = 128
_H1 = 640
_H2 = 320
_DOUT = 128

_NC = 2
_NS = 16
_C = 20
_EPT = _E // (_NC * _NS)
_NCH = _EPT // _C
_NBC = 50
_NRING = 5
_NSB = _NCH // _NBC
_RPT = _NPAD // _NS
_DC = 128

_F32 = jnp.float32


def _make_sc_segsum(ncc):
    mesh = plsc.VectorSubcoreMesh(core_axis_name="c", subcore_axis_name="s",
                                  num_cores=_NC, num_subcores=_NS)

    def body(t_hbm, src_hbm, dst_hbm, out_hbm,
             src_v, dst_v, rows0, rows1, rows2, rows3, rows4, acc,
             sem0, sem1, sem2, sem3, sem4):
        cid = jax.lax.axis_index("c")
        sid = jax.lax.axis_index("s")

        rows = [rows0, rows1, rows2, rows3, rows4]
        sems = [sem0, sem1, sem2, sem3, sem4]

        def zb(k, carry):
            r = k // (_DC // 16)
            col = (k % (_DC // 16)) * 16
            rows0[r, pl.ds(col, 16)] = jnp.zeros((16,), _F32)
            return carry

        for cc in range(ncc):
            jax.lax.fori_loop(0, _C * (_DC // 16), zb, 0)
            for k in range(_RPT // _C):
                pltpu.sync_copy(rows0, acc.at[pl.ds(sid * _RPT + k * _C, _C)])
            plsc.subcore_barrier()

            tc = t_hbm.at[cc]
            for sb in range(_NSB):
                pltpu.sync_copy(src_hbm.at[cid, sid, sb], src_v)
                pltpu.sync_copy(dst_hbm.at[cid, sid, sb], dst_v)
                for b in range(_NRING):
                    pltpu.async_copy(tc.at[src_v.at[b]], rows[b], sems[b])

                def group(g, carry):
                    for b in range(_NRING):
                        j = _NRING * g + b
                        pltpu.make_async_copy(tc.at[src_v.at[j]], rows[b],
                                              sems[b]).wait()
                        pltpu.sync_copy(rows[b], acc.at[dst_v.at[j]],
                                        add=True)

                        @pl.when(j + _NRING < _NBC)
                        def _():
                            pltpu.async_copy(tc.at[src_v.at[j + _NRING]],
                                             rows[b], sems[b])
                    return carry

                jax.lax.fori_loop(0, _NBC // _NRING, group, 0)

            plsc.subcore_barrier()
            pltpu.sync_copy(acc.at[pl.ds(sid * _RPT, _RPT)],
                            out_hbm.at[cc, cid, pl.ds(sid * _RPT, _RPT)])

    return pl.kernel(
        body,
        out_type=jax.ShapeDtypeStruct((ncc, _NC, _NPAD, _DC), _F32),
        mesh=mesh,
        scratch_types=[
            pltpu.VMEM((_NBC, _C), jnp.int32),
            pltpu.VMEM((_NBC, _C), jnp.int32),
            pltpu.VMEM((_C, _DC), _F32),
            pltpu.VMEM((_C, _DC), _F32),
            pltpu.VMEM((_C, _DC), _F32),
            pltpu.VMEM((_C, _DC), _F32),
            pltpu.VMEM((_C, _DC), _F32),
            pltpu.VMEM_SHARED((_NPAD, _DC), _F32),
            pltpu.SemaphoreType.DMA,
            pltpu.SemaphoreType.DMA,
            pltpu.SemaphoreType.DMA,
            pltpu.SemaphoreType.DMA,
            pltpu.SemaphoreType.DMA,
        ],
    )


_BM = 512
_GRID = (_NPAD // _BM,)


def _elu(x):
    return jnp.where(x > 0, x, jnp.exp(jnp.minimum(x, 0.0)) - 1.0)


def _dot(a, b):
    return jnp.dot(a, b, preferred_element_type=_F32)


def _full(shape):
    nd = len(shape)
    return pl.BlockSpec(shape, lambda i: (0,) * nd)


def _rows(d):
    return pl.BlockSpec((_BM, d), lambda i: (i, 0))


def _agg(ncc):
    return pl.BlockSpec((ncc, _NC, _BM, _DC), lambda i: (0, 0, i, 0))


def _k1_body(f_ref, agg_ref, w1a, b1a, w1b, b1b, w2a, wres, bres,
             u2_ref, res_ref):
    x0 = f_ref[...]
    res_ref[...] = _elu(_dot(x0, wres[...]) + bres[...])
    x = x0 + agg_ref[0, 0] + agg_ref[0, 1]
    z1 = _elu(_dot(x, w1a[...]) + b1a[...])
    x1 = _elu(_elu(_dot(z1, w1b[...]) + b1b[...]))
    u2 = _dot(x1, w2a[...])
    for c in range(3):
        u2_ref[c] = u2[:, c * _DC:(c + 1) * _DC]


def _make_k1(interpret=False):
    return pl.pallas_call(
        _k1_body,
        grid=_GRID,
        in_specs=[_rows(_DIN), _agg(1), _full((_DIN, _H1)), _full((1, _H1)),
                  _full((_H1, _H1)), _full((1, _H1)), _full((_H1, 3 * _DC)),
                  _full((_DIN, _DOUT)), _full((1, _DOUT))],
        out_specs=[pl.BlockSpec((3, _BM, _DC), lambda i: (0, i, 0)),
                   _rows(_DOUT)],
        out_shape=[jax.ShapeDtypeStruct((3, _NPAD, _DC), _F32),
                   jax.ShapeDtypeStruct((_NPAD, _DOUT), _F32)],
        interpret=interpret,
    )


def _k2_body(u2_ref, agg_ref, b2a, w2b, b2b, w3a, u3_ref):
    parts = [u2_ref[c] + agg_ref[c, 0] + agg_ref[c, 1] for c in range(3)]
    z2 = _elu(jnp.concatenate(parts, axis=1)[:, :_H2] + b2a[...])
    x2 = _elu(_elu(_dot(z2, w2b[...]) + b2b[...]))
    u3_ref[...] = _dot(x2, w3a[...])


def _make_k2(interpret=False):
    return pl.pallas_call(
        _k2_body,
        grid=_GRID,
        in_specs=[pl.BlockSpec((3, _BM, _DC), lambda i: (0, i, 0)), _agg(3),
                  _full((1, _H2)), _full((_H2, _H2)), _full((1, _H2)),
                  _full((_H2, _DOUT))],
        out_specs=[_rows(_DOUT)],
        out_shape=[jax.ShapeDtypeStruct((_NPAD, _DOUT), _F32)],
        interpret=interpret,
    )


def _k3_body(u3_ref, agg_ref, b3a, w3b, b3b, out_ref):
    z3 = _elu(u3_ref[...] + agg_ref[0, 0] + agg_ref[0, 1] + b3a[...])
    out_ref[...] = _elu(_dot(z3, w3b[...]) + b3b[...])


def _make_k3(interpret=False):
    return pl.pallas_call(
        _k3_body,
        grid=_GRID,
        in_specs=[_rows(_DOUT), _agg(1), _full((1, _DOUT)),
                  _full((_DOUT, _DOUT)), _full((1, _DOUT))],
        out_specs=[_rows(_DOUT)],
        out_shape=[jax.ShapeDtypeStruct((_NPAD, _DOUT), _F32)],
        interpret=interpret,
    )


def kernel(features, edge_index,
           W1a, b1a, W1b, b1b,
           W2a, b2a, W2b, b2b,
           W3a, b3a, W3b, b3b,
           Wres, bres):
    f = jnp.zeros((_NPAD, _DIN), _F32).at[:_N].set(features)
    src = edge_index[0].astype(jnp.int32).reshape(_NC, _NS, _NSB, _NBC, _C)
    dst = edge_index[1].astype(jnp.int32).reshape(_NC, _NS, _NSB, _NBC, _C)
    w2a_pad = jnp.zeros((_H1, 3 * _DC), _F32).at[:, :_H2].set(W2a)

    agg1 = _make_sc_segsum(1)(f.reshape(1, _NPAD, _DIN), src, dst)
    u2, res = _make_k1()(f, agg1, W1a, b1a.reshape(1, -1),
                         W1b, b1b.reshape(1, -1), w2a_pad,
                         Wres, bres.reshape(1, -1))
    agg2 = _make_sc_segsum(3)(u2, src, dst)
    (u3,) = _make_k2()(u2, agg2, b2a.reshape(1, -1),
                       W2b, b2b.reshape(1, -1), W3a)
    agg3 = _make_sc_segsum(1)(u3.reshape(1, _NPAD, _DOUT), src, dst)
    (out,) = _make_k3()(u3, agg3, b3a.reshape(1, -1),
                        W3b, b3b.reshape(1, -1))
    return (out[:_N], res[:_N])


# speedup vs baseline: 1.0587x; 1.0178x over previous
import jax
import jax.numpy as jnp
from jax.experimental import pallas as pl
from jax.experimental.pallas import tpu as pltpu
from jax.experimental.pallas import tpu_sc as plsc

_N = 10000
_NPAD = 10240
_E = 320000
_DIN = 128
_H1 = 640
_H2 = 320
_DOUT = 128

_NC = 2
_NS = 16
_C = 20
_EPT = _E // (_NC * _NS)
_NCH = _EPT // _C
_NBC = 50
_NRING = 5
_NSB = _NCH // _NBC
_RPT = _NPAD // _NS
_DC = 128

_F32 = jnp.float32


def _make_sc_segsum(ncc):
    mesh = plsc.VectorSubcoreMesh(core_axis_name="c", subcore_axis_name="s",
                                  num_cores=_NC, num_subcores=_NS)

    def body(t_hbm, src_hbm, dst_hbm, out_hbm,
             src_v, dst_v, rows0, rows1, rows2, rows3, rows4, acc,
             sem0, sem1, sem2, sem3, sem4):
        cid = jax.lax.axis_index("c")
        sid = jax.lax.axis_index("s")

        rows = [rows0, rows1, rows2, rows3, rows4]
        sems = [sem0, sem1, sem2, sem3, sem4]

        def zb(k, carry):
            r = k // (_DC // 16)
            col = (k % (_DC // 16)) * 16
            rows0[r, pl.ds(col, 16)] = jnp.zeros((16,), _F32)
            return carry

        for cc in range(ncc):
            jax.lax.fori_loop(0, _C * (_DC // 16), zb, 0)
            for k in range(_RPT // _C):
                pltpu.sync_copy(rows0, acc.at[pl.ds(sid * _RPT + k * _C, _C)])
            plsc.subcore_barrier()

            tc = t_hbm.at[cc]
            for sb in range(_NSB):
                pltpu.sync_copy(src_hbm.at[cid, sid, sb], src_v)
                pltpu.sync_copy(dst_hbm.at[cid, sid, sb], dst_v)
                for b in range(_NRING):
                    pltpu.async_copy(tc.at[src_v.at[b]], rows[b], sems[b])

                def group(g, carry):
                    for b in range(_NRING):
                        j = _NRING * g + b
                        pltpu.make_async_copy(tc.at[src_v.at[j]], rows[b],
                                              sems[b]).wait()
                        pltpu.sync_copy(rows[b], acc.at[dst_v.at[j]],
                                        add=True)

                        @pl.when(j + _NRING < _NBC)
                        def _():
                            pltpu.async_copy(tc.at[src_v.at[j + _NRING]],
                                             rows[b], sems[b])
                    return carry

                jax.lax.fori_loop(0, _NBC // _NRING, group, 0)

            plsc.subcore_barrier()
            pltpu.sync_copy(acc.at[pl.ds(sid * _RPT, _RPT)],
                            out_hbm.at[cc, cid, pl.ds(sid * _RPT, _RPT)])

    return pl.kernel(
        body,
        out_type=jax.ShapeDtypeStruct((ncc, _NC, _NPAD, _DC), _F32),
        mesh=mesh,
        scratch_types=[
            pltpu.VMEM((_NBC, _C), jnp.int32),
            pltpu.VMEM((_NBC, _C), jnp.int32),
            pltpu.VMEM((_C, _DC), _F32),
            pltpu.VMEM((_C, _DC), _F32),
            pltpu.VMEM((_C, _DC), _F32),
            pltpu.VMEM((_C, _DC), _F32),
            pltpu.VMEM((_C, _DC), _F32),
            pltpu.VMEM_SHARED((_NPAD, _DC), _F32),
            pltpu.SemaphoreType.DMA,
            pltpu.SemaphoreType.DMA,
            pltpu.SemaphoreType.DMA,
            pltpu.SemaphoreType.DMA,
            pltpu.SemaphoreType.DMA,
        ],
    )


_BM = 1024
_GRID = (_NPAD // _BM,)


def _elu(x):
    return jnp.where(x > 0, x, jnp.exp(jnp.minimum(x, 0.0)) - 1.0)


def _dot(a, b):
    return jnp.dot(a, b, preferred_element_type=_F32)


def _full(shape):
    nd = len(shape)
    return pl.BlockSpec(shape, lambda i: (0,) * nd)


def _rows(d):
    return pl.BlockSpec((_BM, d), lambda i: (i, 0))


def _agg(ncc):
    return pl.BlockSpec((ncc, _NC, _BM, _DC), lambda i: (0, 0, i, 0))


def _k1_body(f_ref, agg_ref, w1a, b1a, w1b, b1b, w2a, wres, bres,
             u2_ref, res_ref):
    x0 = f_ref[...]
    res_ref[...] = _elu(_dot(x0, wres[...]) + bres[...])
    x = x0 + agg_ref[0, 0] + agg_ref[0, 1]
    z1 = _elu(_dot(x, w1a[...]) + b1a[...])
    x1 = _elu(_elu(_dot(z1, w1b[...]) + b1b[...]))
    u2 = _dot(x1, w2a[...])
    for c in range(3):
        u2_ref[c] = u2[:, c * _DC:(c + 1) * _DC]


def _make_k1(interpret=False):
    return pl.pallas_call(
        _k1_body,
        grid=_GRID,
        in_specs=[_rows(_DIN), _agg(1), _full((_DIN, _H1)), _full((1, _H1)),
                  _full((_H1, _H1)), _full((1, _H1)), _full((_H1, 3 * _DC)),
                  _full((_DIN, _DOUT)), _full((1, _DOUT))],
        out_specs=[pl.BlockSpec((3, _BM, _DC), lambda i: (0, i, 0)),
                   _rows(_DOUT)],
        out_shape=[jax.ShapeDtypeStruct((3, _NPAD, _DC), _F32),
                   jax.ShapeDtypeStruct((_NPAD, _DOUT), _F32)],
        interpret=interpret,
    )


def _k2_body(u2_ref, agg_ref, b2a, w2b, b2b, w3a, u3_ref):
    parts = [u2_ref[c] + agg_ref[c, 0] + agg_ref[c, 1] for c in range(3)]
    z2 = _elu(jnp.concatenate(parts, axis=1)[:, :_H2] + b2a[...])
    x2 = _elu(_elu(_dot(z2, w2b[...]) + b2b[...]))
    u3_ref[...] = _dot(x2, w3a[...])


def _make_k2(interpret=False):
    return pl.pallas_call(
        _k2_body,
        grid=_GRID,
        in_specs=[pl.BlockSpec((3, _BM, _DC), lambda i: (0, i, 0)), _agg(3),
                  _full((1, _H2)), _full((_H2, _H2)), _full((1, _H2)),
                  _full((_H2, _DOUT))],
        out_specs=[_rows(_DOUT)],
        out_shape=[jax.ShapeDtypeStruct((_NPAD, _DOUT), _F32)],
        interpret=interpret,
    )


def _k3_body(u3_ref, agg_ref, b3a, w3b, b3b, out_ref):
    z3 = _elu(u3_ref[...] + agg_ref[0, 0] + agg_ref[0, 1] + b3a[...])
    out_ref[...] = _elu(_dot(z3, w3b[...]) + b3b[...])


def _make_k3(interpret=False):
    return pl.pallas_call(
        _k3_body,
        grid=_GRID,
        in_specs=[_rows(_DOUT), _agg(1), _full((1, _DOUT)),
                  _full((_DOUT, _DOUT)), _full((1, _DOUT))],
        out_specs=[_rows(_DOUT)],
        out_shape=[jax.ShapeDtypeStruct((_NPAD, _DOUT), _F32)],
        interpret=interpret,
    )


def kernel(features, edge_index,
           W1a, b1a, W1b, b1b,
           W2a, b2a, W2b, b2b,
           W3a, b3a, W3b, b3b,
           Wres, bres):
    f = jnp.zeros((_NPAD, _DIN), _F32).at[:_N].set(features)
    src = edge_index[0].astype(jnp.int32).reshape(_NC, _NS, _NSB, _NBC, _C)
    dst = edge_index[1].astype(jnp.int32).reshape(_NC, _NS, _NSB, _NBC, _C)
    w2a_pad = jnp.zeros((_H1, 3 * _DC), _F32).at[:, :_H2].set(W2a)

    agg1 = _make_sc_segsum(1)(f.reshape(1, _NPAD, _DIN), src, dst)
    u2, res = _make_k1()(f, agg1, W1a, b1a.reshape(1, -1),
                         W1b, b1b.reshape(1, -1), w2a_pad,
                         Wres, bres.reshape(1, -1))
    agg2 = _make_sc_segsum(3)(u2, src, dst)
    (u3,) = _make_k2()(u2, agg2, b2a.reshape(1, -1),
                       W2b, b2b.reshape(1, -1), W3a)
    agg3 = _make_sc_segsum(1)(u3.reshape(1, _NPAD, _DOUT), src, dst)
    (out,) = _make_k3()(u3, agg3, b3a.reshape(1, -1),
                        W3b, b3b.reshape(1, -1))
    return (out[:_N], res[:_N])


# speedup vs baseline: 1.0617x; 1.0029x over previous
import jax
import jax.numpy as jnp
from jax.experimental import pallas as pl
from jax.experimental.pallas import tpu as pltpu
from jax.experimental.pallas import tpu_sc as plsc

_N = 10000
_NPAD = 10240
_E = 320000
_DIN = 128
_H1 = 640
_H2 = 320
_DOUT = 128

_NC = 2
_NS = 16
_C = 20
_EPT = _E // (_NC * _NS)
_NCH = _EPT // _C
_NBC = 50
_NRING = 5
_NSB = _NCH // _NBC
_RPT = _NPAD // _NS
_DC = 128

_F32 = jnp.float32


def _make_sc_segsum(ncc):
    mesh = plsc.VectorSubcoreMesh(core_axis_name="c", subcore_axis_name="s",
                                  num_cores=_NC, num_subcores=_NS)

    def body(t_hbm, src_hbm, dst_hbm, out_hbm,
             src_v, dst_v, rows0, rows1, rows2, rows3, rows4, acc,
             sem0, sem1, sem2, sem3, sem4):
        cid = jax.lax.axis_index("c")
        sid = jax.lax.axis_index("s")

        rows = [rows0, rows1, rows2, rows3, rows4]
        sems = [sem0, sem1, sem2, sem3, sem4]

        def zb(k, carry):
            r = k // (_DC // 16)
            col = (k % (_DC // 16)) * 16
            rows0[r, pl.ds(col, 16)] = jnp.zeros((16,), _F32)
            return carry

        for cc in range(ncc):
            jax.lax.fori_loop(0, _C * (_DC // 16), zb, 0)
            for k in range(_RPT // _C):
                pltpu.sync_copy(rows0, acc.at[pl.ds(sid * _RPT + k * _C, _C)])
            plsc.subcore_barrier()

            tc = t_hbm.at[cc]
            for sb in range(_NSB):
                pltpu.sync_copy(src_hbm.at[cid, sid, sb], src_v)
                pltpu.sync_copy(dst_hbm.at[cid, sid, sb], dst_v)
                for b in range(_NRING):
                    pltpu.async_copy(tc.at[src_v.at[b]], rows[b], sems[b])

                def group(g, carry):
                    for b in range(_NRING):
                        j = _NRING * g + b
                        pltpu.make_async_copy(tc.at[src_v.at[j]], rows[b],
                                              sems[b]).wait()
                        pltpu.sync_copy(rows[b], acc.at[dst_v.at[j]],
                                        add=True)

                        @pl.when(j + _NRING < _NBC)
                        def _():
                            pltpu.async_copy(tc.at[src_v.at[j + _NRING]],
                                             rows[b], sems[b])
                    return carry

                jax.lax.fori_loop(0, _NBC // _NRING, group, 0)

            plsc.subcore_barrier()
            pltpu.sync_copy(acc.at[pl.ds(sid * _RPT, _RPT)],
                            out_hbm.at[cc, cid, pl.ds(sid * _RPT, _RPT)])

    return pl.kernel(
        body,
        out_type=jax.ShapeDtypeStruct((ncc, _NC, _NPAD, _DC), _F32),
        mesh=mesh,
        scratch_types=[
            pltpu.VMEM((_NBC, _C), jnp.int32),
            pltpu.VMEM((_NBC, _C), jnp.int32),
            pltpu.VMEM((_C, _DC), _F32),
            pltpu.VMEM((_C, _DC), _F32),
            pltpu.VMEM((_C, _DC), _F32),
            pltpu.VMEM((_C, _DC), _F32),
            pltpu.VMEM((_C, _DC), _F32),
            pltpu.VMEM_SHARED((_NPAD, _DC), _F32),
            pltpu.SemaphoreType.DMA,
            pltpu.SemaphoreType.DMA,
            pltpu.SemaphoreType.DMA,
            pltpu.SemaphoreType.DMA,
            pltpu.SemaphoreType.DMA,
        ],
    )


_BM = 2048
_GRID = (_NPAD // _BM,)


def _elu(x):
    return jnp.where(x > 0, x, jnp.exp(jnp.minimum(x, 0.0)) - 1.0)


def _dot(a, b):
    return jnp.dot(a, b, preferred_element_type=_F32)


def _full(shape):
    nd = len(shape)
    return pl.BlockSpec(shape, lambda i: (0,) * nd)


def _rows(d):
    return pl.BlockSpec((_BM, d), lambda i: (i, 0))


def _agg(ncc):
    return pl.BlockSpec((ncc, _NC, _BM, _DC), lambda i: (0, 0, i, 0))


def _k1_body(f_ref, agg_ref, w1a, b1a, w1b, b1b, w2a, wres, bres,
             u2_ref, res_ref):
    x0 = f_ref[...]
    res_ref[...] = _elu(_dot(x0, wres[...]) + bres[...])
    x = x0 + agg_ref[0, 0] + agg_ref[0, 1]
    z1 = _elu(_dot(x, w1a[...]) + b1a[...])
    x1 = _elu(_elu(_dot(z1, w1b[...]) + b1b[...]))
    u2 = _dot(x1, w2a[...])
    for c in range(3):
        u2_ref[c] = u2[:, c * _DC:(c + 1) * _DC]


def _make_k1(interpret=False):
    return pl.pallas_call(
        _k1_body,
        grid=_GRID,
        in_specs=[_rows(_DIN), _agg(1), _full((_DIN, _H1)), _full((1, _H1)),
                  _full((_H1, _H1)), _full((1, _H1)), _full((_H1, 3 * _DC)),
                  _full((_DIN, _DOUT)), _full((1, _DOUT))],
        out_specs=[pl.BlockSpec((3, _BM, _DC), lambda i: (0, i, 0)),
                   _rows(_DOUT)],
        out_shape=[jax.ShapeDtypeStruct((3, _NPAD, _DC), _F32),
                   jax.ShapeDtypeStruct((_NPAD, _DOUT), _F32)],
        interpret=interpret,
    )


def _k2_body(u2_ref, agg_ref, b2a, w2b, b2b, w3a, u3_ref):
    parts = [u2_ref[c] + agg_ref[c, 0] + agg_ref[c, 1] for c in range(3)]
    z2 = _elu(jnp.concatenate(parts, axis=1)[:, :_H2] + b2a[...])
    x2 = _elu(_elu(_dot(z2, w2b[...]) + b2b[...]))
    u3_ref[...] = _dot(x2, w3a[...])


def _make_k2(interpret=False):
    return pl.pallas_call(
        _k2_body,
        grid=_GRID,
        in_specs=[pl.BlockSpec((3, _BM, _DC), lambda i: (0, i, 0)), _agg(3),
                  _full((1, _H2)), _full((_H2, _H2)), _full((1, _H2)),
                  _full((_H2, _DOUT))],
        out_specs=[_rows(_DOUT)],
        out_shape=[jax.ShapeDtypeStruct((_NPAD, _DOUT), _F32)],
        interpret=interpret,
    )


def _k3_body(u3_ref, agg_ref, b3a, w3b, b3b, out_ref):
    z3 = _elu(u3_ref[...] + agg_ref[0, 0] + agg_ref[0, 1] + b3a[...])
    out_ref[...] = _elu(_dot(z3, w3b[...]) + b3b[...])


def _make_k3(interpret=False):
    return pl.pallas_call(
        _k3_body,
        grid=_GRID,
        in_specs=[_rows(_DOUT), _agg(1), _full((1, _DOUT)),
                  _full((_DOUT, _DOUT)), _full((1, _DOUT))],
        out_specs=[_rows(_DOUT)],
        out_shape=[jax.ShapeDtypeStruct((_NPAD, _DOUT), _F32)],
        interpret=interpret,
    )


def kernel(features, edge_index,
           W1a, b1a, W1b, b1b,
           W2a, b2a, W2b, b2b,
           W3a, b3a, W3b, b3b,
           Wres, bres):
    f = jnp.zeros((_NPAD, _DIN), _F32).at[:_N].set(features)
    src = edge_index[0].astype(jnp.int32).reshape(_NC, _NS, _NSB, _NBC, _C)
    dst = edge_index[1].astype(jnp.int32).reshape(_NC, _NS, _NSB, _NBC, _C)
    w2a_pad = jnp.zeros((_H1, 3 * _DC), _F32).at[:, :_H2].set(W2a)

    agg1 = _make_sc_segsum(1)(f.reshape(1, _NPAD, _DIN), src, dst)
    u2, res = _make_k1()(f, agg1, W1a, b1a.reshape(1, -1),
                         W1b, b1b.reshape(1, -1), w2a_pad,
                         Wres, bres.reshape(1, -1))
    agg2 = _make_sc_segsum(3)(u2, src, dst)
    (u3,) = _make_k2()(u2, agg2, b2a.reshape(1, -1),
                       W2b, b2b.reshape(1, -1), W3a)
    agg3 = _make_sc_segsum(1)(u3.reshape(1, _NPAD, _DOUT), src, dst)
    (out,) = _make_k3()(u3, agg3, b3a.reshape(1, -1),
                        W3b, b3b.reshape(1, -1))
    return (out[:_N], res[:_N])
